# baseline (speedup 1.0000x reference)
import functools

import jax
import jax.numpy as jnp
from jax import lax
from jax.experimental import pallas as pl
from jax.experimental.pallas import tpu as pltpu
from jax.experimental.pallas import tpu_sc as plsc

N = 10000
H = 64
K = 20
NLAYERS = 4

_knn_S = 79
NP = _knn_S * 128

NW = 32
CH = 128
NCH = 50
BPW = CH * NCH
E_PAD = NW * BPW


def _elu(x):
    return jnp.where(x > 0, x, jnp.exp(jnp.minimum(x, 0.0)) - 1.0)


def _mlp2_body(x_ref, w1_ref, b1_ref, w2_ref, b2_ref, o_ref):
    t = _elu(jnp.dot(x_ref[...], w1_ref[...],
                     preferred_element_type=jnp.float32) + b1_ref[...])
    o_ref[...] = _elu(jnp.dot(t, w2_ref[...],
                              preferred_element_type=jnp.float32) + b2_ref[...])


def _encode(x, w1, b1, w2, b2):
    R = 2000
    return pl.pallas_call(
        _mlp2_body,
        grid=(N // R,),
        in_specs=[pl.BlockSpec((R, 8), lambda i: (i, 0)),
                  pl.BlockSpec((8, H), lambda i: (0, 0)),
                  pl.BlockSpec((1, H), lambda i: (0, 0)),
                  pl.BlockSpec((H, H), lambda i: (0, 0)),
                  pl.BlockSpec((1, H), lambda i: (0, 0))],
        out_specs=pl.BlockSpec((R, H), lambda i: (i, 0)),
        out_shape=jax.ShapeDtypeStruct((N, H), jnp.float32),
    )(x, w1, b1.reshape(1, H), w2, b2.reshape(1, H))


def _out_body(h_ref, w1_ref, b1_ref, w2_ref, b2_ref, w3_ref, b3_ref, o_ref):
    t = _elu(jnp.dot(h_ref[...], w1_ref[...],
                     preferred_element_type=jnp.float32) + b1_ref[...])
    t = _elu(jnp.dot(t, w2_ref[...],
                     preferred_element_type=jnp.float32) + b2_ref[...])
    o_ref[...] = jnp.dot(t, w3_ref[...],
                         preferred_element_type=jnp.float32) + b3_ref[...]


def _outmlp(h, w1, b1, w2, b2, w3, b3):
    R = 2000
    return pl.pallas_call(
        _out_body,
        grid=(N // R,),
        in_specs=[pl.BlockSpec((R, H), lambda i: (i, 0)),
                  pl.BlockSpec((H, 64), lambda i: (0, 0)),
                  pl.BlockSpec((1, 64), lambda i: (0, 0)),
                  pl.BlockSpec((64, 32), lambda i: (0, 0)),
                  pl.BlockSpec((1, 32), lambda i: (0, 0)),
                  pl.BlockSpec((32, 8), lambda i: (0, 0)),
                  pl.BlockSpec((1, 8), lambda i: (0, 0))],
        out_specs=pl.BlockSpec((R, 8), lambda i: (i, 0)),
        out_shape=jax.ShapeDtypeStruct((N, 8), jnp.float32),
    )(h, w1, b1.reshape(1, 64), w2, b2.reshape(1, 32), w3, b3.reshape(1, 8))


def _knn_body(ht_ref, hT_ref, wd_ref, w1b_ref, b1_ref, idx_ref, a_ref, g_ref):
    ht = ht_ref[...]
    hT = hT_ref[...]
    a_ref[...] = jnp.dot(ht, wd_ref[...],
                         preferred_element_type=jnp.float32) + b1_ref[...]
    g_ref[...] = jnp.dot(ht, w1b_ref[...],
                         preferred_element_type=jnp.float32)
    sq_row = jnp.sum(hT * hT, axis=0, keepdims=True)
    sq_t = jnp.sum(ht * ht, axis=1, keepdims=True)
    prod = jnp.dot(ht, hT, preferred_element_type=jnp.float32)
    d = sq_t - 2.0 * prod + sq_row
    iota = lax.broadcasted_iota(jnp.int32, d.shape, 1)
    d = jnp.where(iota < N, d, jnp.inf)

    cols = []
    for _ in range(K):
        sel = jnp.argmin(d, axis=1)[:, None]
        cols.append(sel)
        d = jnp.where(iota == sel, jnp.inf, d)
    idx_ref[...] = jnp.concatenate(cols, axis=1)


def _knn(h, hT, wd, w1b, b1):
    R = 400
    return pl.pallas_call(
        _knn_body,
        grid=(N // R,),
        in_specs=[pl.BlockSpec((R, H), lambda i: (i, 0)),
                  pl.BlockSpec((H, NP), lambda i: (0, 0)),
                  pl.BlockSpec((H, H), lambda i: (0, 0)),
                  pl.BlockSpec((H, H), lambda i: (0, 0)),
                  pl.BlockSpec((1, H), lambda i: (0, 0))],
        out_specs=[pl.BlockSpec((R, K), lambda i: (i, 0)),
                   pl.BlockSpec((R, H), lambda i: (i, 0)),
                   pl.BlockSpec((R, H), lambda i: (i, 0))],
        out_shape=[jax.ShapeDtypeStruct((N, K), jnp.int32),
                   jax.ShapeDtypeStruct((N, H), jnp.float32),
                   jax.ShapeDtypeStruct((N, H), jnp.float32)],
    )(h, hT, wd, w1b, b1.reshape(1, H))


def _gather(table, idx_pad):
    mesh = plsc.VectorSubcoreMesh(core_axis_name="c", subcore_axis_name="s")

    @functools.partial(
        pl.kernel,
        mesh=mesh,
        out_type=jax.ShapeDtypeStruct((E_PAD, H), jnp.float32),
        scratch_types=[pltpu.VMEM((BPW,), jnp.int32),
                       pltpu.VMEM((CH, H), jnp.float32),
                       pltpu.VMEM((CH, H), jnp.float32),
                       pltpu.SemaphoreType.DMA,
                       pltpu.SemaphoreType.DMA],
        compiler_params=pltpu.CompilerParams(use_tc_tiling_on_sc=False),
    )
    def k(table_hbm, idx_hbm, out_hbm, idx_v, rows0, rows1, sem0, sem1):
        wid = lax.axis_index("s") * 2 + lax.axis_index("c")
        base = wid * BPW
        pltpu.sync_copy(idx_hbm.at[pl.ds(base, BPW)], idx_v)

        def gather(c, buf, sem):
            return pltpu.async_copy(
                table_hbm.at[idx_v.at[pl.ds(c * CH, CH)]], buf, sem)

        def gwait(c, buf, sem):
            pltpu.make_async_copy(
                table_hbm.at[idx_v.at[pl.ds(c * CH, CH)]], buf, sem).wait()

        gather(0, rows0, sem0)

        def body(t, carry):
            c = t * 2
            gather(c + 1, rows1, sem1)
            gwait(c, rows0, sem0)
            pltpu.sync_copy(rows0, out_hbm.at[pl.ds(base + c * CH, CH)])

            @pl.when(t + 1 < NCH // 2)
            def _():
                gather(c + 2, rows0, sem0)

            gwait(c + 1, rows1, sem1)
            pltpu.sync_copy(rows1, out_hbm.at[pl.ds(base + (c + 1) * CH, CH)])
            return carry

        lax.fori_loop(0, NCH // 2, body, 0)

    return k(table, idx_pad)


def _agg_body(a_ref, g3_ref, w2_ref, b2_ref, o_ref):
    a = a_ref[...]
    r = a.shape[0]
    ms = [_elu(a + g3_ref[:, j, :]) for j in range(K)]
    m = jnp.concatenate(ms, axis=0)
    mm = jnp.dot(m, w2_ref[...], preferred_element_type=jnp.float32)
    best = mm[0:r]
    for j in range(1, K):
        best = jnp.maximum(best, mm[j * r:(j + 1) * r])
    o_ref[...] = _elu(best + b2_ref[...])


def _agg(a, g3, w2, b2):
    R = 400
    return pl.pallas_call(
        _agg_body,
        grid=(N // R,),
        in_specs=[pl.BlockSpec((R, H), lambda i: (i, 0)),
                  pl.BlockSpec((R, K, H), lambda i: (i, 0, 0)),
                  pl.BlockSpec((H, H), lambda i: (0, 0)),
                  pl.BlockSpec((1, H), lambda i: (0, 0))],
        out_specs=pl.BlockSpec((R, H), lambda i: (i, 0)),
        out_shape=jax.ShapeDtypeStruct((N, H), jnp.float32),
    )(a, g3, w2, b2.reshape(1, H))


def kernel(x, params, batch):
    p = params
    h = _encode(x, p['lc_w1'], p['lc_b1'], p['lc_w2'], p['lc_b2'])
    for i in range(NLAYERS):
        w1 = p['c%d_w1' % i]
        b1 = p['c%d_b1' % i]
        w2 = p['c%d_w2' % i]
        b2 = p['c%d_b2' % i]
        wd = w1[:H] - w1[H:]
        w1b = w1[H:]
        hTp = jnp.concatenate(
            [h.T, jnp.zeros((H, NP - N), jnp.float32)], axis=1)
        idx, a, g = _knn(h, hTp, wd, w1b, b1)
        idx_pad = jnp.concatenate(
            [idx.reshape(-1), jnp.zeros((E_PAD - N * K,), jnp.int32)])
        gp = _gather(g, idx_pad)
        g3 = gp.reshape(E_PAD // K, K, H)
        h = _agg(a, g3, w2, b2)
    o = _outmlp(h, p['o_w1'], p['o_b1'], p['o_w2'], p['o_b2'],
                p['o_w3'], p['o_b3'])
    return (o, batch)


# speedup vs baseline: 1.0090x; 1.0090x over previous
import functools

import jax
import jax.numpy as jnp
from jax import lax
from jax.experimental import pallas as pl
from jax.experimental.pallas import tpu as pltpu
from jax.experimental.pallas import tpu_sc as plsc

N = 10000
H = 64
K = 20
NLAYERS = 4

_knn_S = 79
NP = _knn_S * 128

NW = 32
CH = 128
NCH = 50
BPW = CH * NCH
E_PAD = NW * BPW


def _elu(x):
    return jnp.where(x > 0, x, jnp.exp(jnp.minimum(x, 0.0)) - 1.0)


def _mlp2_body(x_ref, w1_ref, b1_ref, w2_ref, b2_ref, o_ref):
    t = _elu(jnp.dot(x_ref[...], w1_ref[...],
                     preferred_element_type=jnp.float32) + b1_ref[...])
    o_ref[...] = _elu(jnp.dot(t, w2_ref[...],
                              preferred_element_type=jnp.float32) + b2_ref[...])


def _encode(x, w1, b1, w2, b2):
    R = 2000
    return pl.pallas_call(
        _mlp2_body,
        grid=(N // R,),
        in_specs=[pl.BlockSpec((R, 8), lambda i: (i, 0)),
                  pl.BlockSpec((8, H), lambda i: (0, 0)),
                  pl.BlockSpec((1, H), lambda i: (0, 0)),
                  pl.BlockSpec((H, H), lambda i: (0, 0)),
                  pl.BlockSpec((1, H), lambda i: (0, 0))],
        out_specs=pl.BlockSpec((R, H), lambda i: (i, 0)),
        out_shape=jax.ShapeDtypeStruct((N, H), jnp.float32),
    )(x, w1, b1.reshape(1, H), w2, b2.reshape(1, H))


def _out_body(h_ref, w1_ref, b1_ref, w2_ref, b2_ref, w3_ref, b3_ref, o_ref):
    t = _elu(jnp.dot(h_ref[...], w1_ref[...],
                     preferred_element_type=jnp.float32) + b1_ref[...])
    t = _elu(jnp.dot(t, w2_ref[...],
                     preferred_element_type=jnp.float32) + b2_ref[...])
    o_ref[...] = jnp.dot(t, w3_ref[...],
                         preferred_element_type=jnp.float32) + b3_ref[...]


def _outmlp(h, w1, b1, w2, b2, w3, b3):
    R = 2000
    return pl.pallas_call(
        _out_body,
        grid=(N // R,),
        in_specs=[pl.BlockSpec((R, H), lambda i: (i, 0)),
                  pl.BlockSpec((H, 64), lambda i: (0, 0)),
                  pl.BlockSpec((1, 64), lambda i: (0, 0)),
                  pl.BlockSpec((64, 32), lambda i: (0, 0)),
                  pl.BlockSpec((1, 32), lambda i: (0, 0)),
                  pl.BlockSpec((32, 8), lambda i: (0, 0)),
                  pl.BlockSpec((1, 8), lambda i: (0, 0))],
        out_specs=pl.BlockSpec((R, 8), lambda i: (i, 0)),
        out_shape=jax.ShapeDtypeStruct((N, 8), jnp.float32),
    )(h, w1, b1.reshape(1, 64), w2, b2.reshape(1, 32), w3, b3.reshape(1, 8))


def _prep_body(h_ref, wd_ref, w1b_ref, b1_ref, a_ref, g_ref):
    h = h_ref[...]
    a_ref[...] = jnp.dot(h, wd_ref[...],
                         preferred_element_type=jnp.float32) + b1_ref[...]
    g_ref[...] = jnp.dot(h, w1b_ref[...],
                         preferred_element_type=jnp.float32)


def _prep(h, wd, w1b, b1):
    R = 2000
    return pl.pallas_call(
        _prep_body,
        grid=(N // R,),
        in_specs=[pl.BlockSpec((R, H), lambda i: (i, 0)),
                  pl.BlockSpec((H, H), lambda i: (0, 0)),
                  pl.BlockSpec((H, H), lambda i: (0, 0)),
                  pl.BlockSpec((1, H), lambda i: (0, 0))],
        out_specs=[pl.BlockSpec((R, H), lambda i: (i, 0)),
                   pl.BlockSpec((R, H), lambda i: (i, 0))],
        out_shape=[jax.ShapeDtypeStruct((N, H), jnp.float32),
                   jax.ShapeDtypeStruct((N, H), jnp.float32)],
    )(h, wd, w1b, b1.reshape(1, H))


def _knn_body(ht_ref, hT_ref, idx_ref):
    ht = ht_ref[...]
    hT = hT_ref[...]
    sq_row = jnp.sum(hT * hT, axis=0, keepdims=True)
    sq_t = jnp.sum(ht * ht, axis=1, keepdims=True)
    prod = jnp.dot(ht, hT, preferred_element_type=jnp.float32)
    d = sq_t - 2.0 * prod + sq_row
    iota = lax.broadcasted_iota(jnp.int32, d.shape, 1)
    d = jnp.where(iota < N, d, jnp.inf)

    cols = []
    for _ in range(K):
        sel = jnp.argmin(d, axis=1)[:, None]
        cols.append(sel)
        d = jnp.where(iota == sel, jnp.inf, d)
    idx_ref[...] = jnp.concatenate(cols, axis=1)


def _knn(h, hT):
    R = 400
    return pl.pallas_call(
        _knn_body,
        grid=(N // R,),
        in_specs=[pl.BlockSpec((R, H), lambda i: (i, 0)),
                  pl.BlockSpec((H, NP), lambda i: (0, 0))],
        out_specs=pl.BlockSpec((R, K), lambda i: (i, 0)),
        out_shape=jax.ShapeDtypeStruct((N, K), jnp.int32),
    )(h, hT)


def _gather(table, idx_pad):
    mesh = plsc.VectorSubcoreMesh(core_axis_name="c", subcore_axis_name="s")

    @functools.partial(
        pl.kernel,
        mesh=mesh,
        out_type=jax.ShapeDtypeStruct((E_PAD, H), jnp.float32),
        scratch_types=[pltpu.VMEM((BPW,), jnp.int32),
                       pltpu.VMEM((CH, H), jnp.float32),
                       pltpu.VMEM((CH, H), jnp.float32),
                       pltpu.SemaphoreType.DMA,
                       pltpu.SemaphoreType.DMA],
        compiler_params=pltpu.CompilerParams(use_tc_tiling_on_sc=False),
    )
    def k(table_hbm, idx_hbm, out_hbm, idx_v, rows0, rows1, sem0, sem1):
        wid = lax.axis_index("s") * 2 + lax.axis_index("c")
        base = wid * BPW
        pltpu.sync_copy(idx_hbm.at[pl.ds(base, BPW)], idx_v)

        def gather(c, buf, sem):
            return pltpu.async_copy(
                table_hbm.at[idx_v.at[pl.ds(c * CH, CH)]], buf, sem)

        def gwait(c, buf, sem):
            pltpu.make_async_copy(
                table_hbm.at[idx_v.at[pl.ds(c * CH, CH)]], buf, sem).wait()

        gather(0, rows0, sem0)

        def body(t, carry):
            c = t * 2
            gather(c + 1, rows1, sem1)
            gwait(c, rows0, sem0)
            pltpu.sync_copy(rows0, out_hbm.at[pl.ds(base + c * CH, CH)])

            @pl.when(t + 1 < NCH // 2)
            def _():
                gather(c + 2, rows0, sem0)

            gwait(c + 1, rows1, sem1)
            pltpu.sync_copy(rows1, out_hbm.at[pl.ds(base + (c + 1) * CH, CH)])
            return carry

        lax.fori_loop(0, NCH // 2, body, 0)

    return k(table, idx_pad)


def _agg_body(a_ref, g3_ref, w2_ref, b2_ref, o_ref):
    a = a_ref[...]
    r = a.shape[0]
    ms = [_elu(a + g3_ref[:, j, :]) for j in range(K)]
    m = jnp.concatenate(ms, axis=0)
    mm = jnp.dot(m, w2_ref[...], preferred_element_type=jnp.float32)
    best = mm[0:r]
    for j in range(1, K):
        best = jnp.maximum(best, mm[j * r:(j + 1) * r])
    o_ref[...] = _elu(best + b2_ref[...])


def _agg(a, g3, w2, b2):
    R = 400
    return pl.pallas_call(
        _agg_body,
        grid=(N // R,),
        in_specs=[pl.BlockSpec((R, H), lambda i: (i, 0)),
                  pl.BlockSpec((R, K, H), lambda i: (i, 0, 0)),
                  pl.BlockSpec((H, H), lambda i: (0, 0)),
                  pl.BlockSpec((1, H), lambda i: (0, 0))],
        out_specs=pl.BlockSpec((R, H), lambda i: (i, 0)),
        out_shape=jax.ShapeDtypeStruct((N, H), jnp.float32),
    )(a, g3, w2, b2.reshape(1, H))


def kernel(x, params, batch):
    p = params
    h = _encode(x, p['lc_w1'], p['lc_b1'], p['lc_w2'], p['lc_b2'])
    for i in range(NLAYERS):
        w1 = p['c%d_w1' % i]
        b1 = p['c%d_b1' % i]
        w2 = p['c%d_w2' % i]
        b2 = p['c%d_b2' % i]
        wd = w1[:H] - w1[H:]
        w1b = w1[H:]
        a, g = _prep(h, wd, w1b, b1)
        hTp = jnp.concatenate(
            [h.T, jnp.zeros((H, NP - N), jnp.float32)], axis=1)
        idx = _knn(h, hTp)
        idx_pad = jnp.concatenate(
            [idx.reshape(-1), jnp.zeros((E_PAD - N * K,), jnp.int32)])
        gp = _gather(g, idx_pad)
        g3 = gp.reshape(E_PAD // K, K, H)
        h = _agg(a, g3, w2, b2)
    o = _outmlp(h, p['o_w1'], p['o_b1'], p['o_w2'], p['o_b2'],
                p['o_w3'], p['o_b3'])
    return (o, batch)
